# baseline (speedup 1.0000x reference)
import functools

import jax
import jax.numpy as jnp
from jax import lax
from jax.experimental import pallas as pl
from jax.experimental.pallas import tpu as pltpu
from jax.experimental.pallas import tpu_sc as plsc


def _build_mask_sc(idx2d, sc2d, L, K, LK=None):
    LK = L if LK is None else LK
    info = plsc.get_sparse_core_info()
    NC, NS, NL = info.num_cores, info.num_subcores, info.num_lanes
    NW = NC * NS
    assert L % NW == 0 and K % NL == 0
    rows_per_w = L // NW
    k_vecs = K // NL

    mesh = plsc.VectorSubcoreMesh(core_axis_name="c", subcore_axis_name="s")

    @functools.partial(
        pl.kernel,
        mesh=mesh,
        out_type=jax.ShapeDtypeStruct((L, LK), jnp.float32),
        scratch_types=[
            pltpu.VMEM((rows_per_w, K), jnp.int32),
            pltpu.VMEM((rows_per_w, K), jnp.float32),
            pltpu.VMEM((LK,), jnp.float32),
        ],
        compiler_params=pltpu.CompilerParams(
            needs_layout_passes=False, use_tc_tiling_on_sc=True),
    )
    def scatter_kernel(idx_hbm, sc_hbm, out_hbm, idx_v, sc_v, buf):
        wid = lax.axis_index("s") * NC + lax.axis_index("c")
        base = wid * rows_per_w
        pltpu.sync_copy(idx_hbm.at[pl.ds(base, rows_per_w)], idx_v)
        pltpu.sync_copy(sc_hbm.at[pl.ds(base, rows_per_w)], sc_v)
        zeros16 = jnp.zeros((NL,), jnp.float32)
        for z in range(LK // NL):
            buf[pl.ds(z * NL, NL)] = zeros16
        for r in range(rows_per_w):
            ivs = []
            for j in range(k_vecs):
                iv = idx_v[r, pl.ds(j * NL, NL)]
                sv = sc_v[r, pl.ds(j * NL, NL)]
                plsc.addupdate_scatter(buf, [iv], sv)
                ivs.append(iv)
            pltpu.sync_copy(buf, out_hbm.at[base + r])
            for iv in ivs:
                plsc.store_scatter(buf, [iv], zeros16)

    return scatter_kernel(idx2d, sc2d)


def _attn_body(q_ref, k_ref, v_ref, m_ref, o_ref, *, D, c):
    h = pl.program_id(1)
    qb = (q_ref[0] * c).astype(jnp.bfloat16)
    kh = k_ref[h].astype(jnp.bfloat16)
    s = jax.lax.dot_general(
        qb, kh, (((0,), (0,)), ((), ())),
        preferred_element_type=jnp.float32)
    p = jnp.exp2(s) * m_ref[...]
    L = kh.shape[1]
    vz = jnp.concatenate(
        [v_ref[h].astype(jnp.bfloat16),
         jnp.ones((8, L), jnp.bfloat16)], axis=0)
    r = jax.lax.dot_general(
        vz, p.astype(jnp.bfloat16), (((1,), (1,)), ((), ())),
        preferred_element_type=jnp.float32)
    o_ref[0] = r[:D] / (r[D:D + 1] + 1e-30)


def _attention_tc(q, k, v, m, BQ=2048):
    H, D, LQ = q.shape
    LK = k.shape[2]
    nq = LQ // BQ
    grid = (nq, H)
    c = (D ** -0.5) * 1.4426950408889634
    return pl.pallas_call(
        functools.partial(_attn_body, D=D, c=c),
        grid=grid,
        in_specs=[
            pl.BlockSpec((1, D, BQ), lambda i, h: (h, 0, i)),
            pl.BlockSpec((H, D, LK), lambda i, h: (0, 0, 0)),
            pl.BlockSpec((H, D, LK), lambda i, h: (0, 0, 0)),
            pl.BlockSpec((BQ, LK), lambda i, h: (i, 0)),
        ],
        out_specs=pl.BlockSpec((1, D, BQ), lambda i, h: (h, 0, i)),
        out_shape=jax.ShapeDtypeStruct((H, D, LQ), jnp.float32),
        compiler_params=pltpu.CompilerParams(
            dimension_semantics=("parallel", "parallel"),
            fuse_transposed_lhs_in_matmul=True,
        ),
    )(q, k, v, m)


def kernel(q, k, v, topk_indices, topk_scores):
    B, H, L, D = q.shape
    K = topk_indices.shape[-1]
    assert B == 1
    idx2d = topk_indices.reshape(L, K).astype(jnp.int32)
    sc2d = topk_scores.reshape(L, K).astype(jnp.float32)
    qt = jnp.swapaxes(q[0], 1, 2)
    kt = jnp.swapaxes(k[0], 1, 2)
    vt = jnp.swapaxes(v[0], 1, 2)
    CL = L // 2
    masks = [_build_mask_sc(idx2d[i * CL:(i + 1) * CL],
                            sc2d[i * CL:(i + 1) * CL], CL, K, LK=L)
             for i in range(2)]
    outs = [_attention_tc(qt[:, :, i * CL:(i + 1) * CL], kt, vt, masks[i],
                          BQ=CL)
            for i in range(2)]
    out_t = jnp.concatenate(outs, axis=2)
    return jnp.swapaxes(out_t, 1, 2)[None]


# speedup vs baseline: 1.0791x; 1.0791x over previous
import functools

import jax
import jax.numpy as jnp
from jax import lax
from jax.experimental import pallas as pl
from jax.experimental.pallas import tpu as pltpu
from jax.experimental.pallas import tpu_sc as plsc


def _build_mask_sc(idx2d, sc2d, L, K):
    info = plsc.get_sparse_core_info()
    NC, NS, NL = info.num_cores, info.num_subcores, info.num_lanes
    NW = NC * NS
    assert L % NW == 0 and K % NL == 0
    rows_per_w = L // NW
    k_vecs = K // NL

    mesh = plsc.VectorSubcoreMesh(core_axis_name="c", subcore_axis_name="s")

    @functools.partial(
        pl.kernel,
        mesh=mesh,
        out_type=jax.ShapeDtypeStruct((L, L), jnp.float32),
        scratch_types=[
            pltpu.VMEM((rows_per_w, K), jnp.int32),
            pltpu.VMEM((rows_per_w, K), jnp.float32),
            pltpu.VMEM((L,), jnp.float32),
        ],
        compiler_params=pltpu.CompilerParams(
            needs_layout_passes=False, use_tc_tiling_on_sc=True),
    )
    def scatter_kernel(idx_hbm, sc_hbm, out_hbm, idx_v, sc_v, buf):
        wid = lax.axis_index("s") * NC + lax.axis_index("c")
        base = wid * rows_per_w
        pltpu.sync_copy(idx_hbm.at[pl.ds(base, rows_per_w)], idx_v)
        pltpu.sync_copy(sc_hbm.at[pl.ds(base, rows_per_w)], sc_v)
        zeros16 = jnp.zeros((NL,), jnp.float32)
        for z in range(L // NL):
            buf[pl.ds(z * NL, NL)] = zeros16
        for r in range(rows_per_w):
            ivs = []
            for j in range(k_vecs):
                iv = idx_v[r, pl.ds(j * NL, NL)]
                sv = sc_v[r, pl.ds(j * NL, NL)]
                plsc.addupdate_scatter(buf, [iv], sv)
                ivs.append(iv)
            pltpu.sync_copy(buf, out_hbm.at[base + r])
            for iv in ivs:
                plsc.store_scatter(buf, [iv], zeros16)

    return scatter_kernel(idx2d, sc2d)


def _attn_body(q_ref, k_ref, v_ref, m_ref, o_ref, *, D, c):
    h = pl.program_id(1)
    qb = (q_ref[0] * c).astype(jnp.bfloat16)
    kh = k_ref[h].astype(jnp.bfloat16)
    s = jax.lax.dot_general(
        qb, kh, (((0,), (0,)), ((), ())),
        preferred_element_type=jnp.float32)
    p = jnp.exp2(s) * m_ref[...]
    L = kh.shape[1]
    vz = jnp.concatenate(
        [v_ref[h].astype(jnp.bfloat16),
         jnp.ones((8, L), jnp.bfloat16)], axis=0)
    r = jax.lax.dot_general(
        vz, p.astype(jnp.bfloat16), (((1,), (1,)), ((), ())),
        preferred_element_type=jnp.float32)
    o_ref[0] = r[:D] / (r[D:D + 1] + 1e-30)


def _attention_tc(q, k, v, m, BQ=2048):
    H, D, L = q.shape
    nq = L // BQ
    grid = (nq, H)
    c = (D ** -0.5) * 1.4426950408889634
    return pl.pallas_call(
        functools.partial(_attn_body, D=D, c=c),
        grid=grid,
        in_specs=[
            pl.BlockSpec((1, D, BQ), lambda i, h: (h, 0, i)),
            pl.BlockSpec((H, D, L), lambda i, h: (0, 0, 0)),
            pl.BlockSpec((H, D, L), lambda i, h: (0, 0, 0)),
            pl.BlockSpec((BQ, L), lambda i, h: (i, 0)),
        ],
        out_specs=pl.BlockSpec((1, D, BQ), lambda i, h: (h, 0, i)),
        out_shape=jax.ShapeDtypeStruct((H, D, L), jnp.float32),
        compiler_params=pltpu.CompilerParams(
            dimension_semantics=("parallel", "parallel"),
            fuse_transposed_lhs_in_matmul=True,
        ),
    )(q, k, v, m)


def kernel(q, k, v, topk_indices, topk_scores):
    B, H, L, D = q.shape
    K = topk_indices.shape[-1]
    assert B == 1
    idx2d = topk_indices.reshape(L, K).astype(jnp.int32)
    sc2d = topk_scores.reshape(L, K).astype(jnp.float32)
    m = _build_mask_sc(idx2d, sc2d, L, K)
    qt = jnp.swapaxes(q[0], 1, 2)
    kt = jnp.swapaxes(k[0], 1, 2)
    vt = jnp.swapaxes(v[0], 1, 2)
    out_t = _attention_tc(qt, kt, vt, m)
    return jnp.swapaxes(out_t, 1, 2)[None]
